# Initial kernel scaffold; baseline (speedup 1.0000x reference)
import jax
import jax.numpy as jnp
from jax.experimental import pallas as pl


def kernel(x, edge_index, W_l1, b_l1, W_r1, W_l2, b_l2, W_r2, W_l3, b_l3, W_r3, W_out, b_out):
    raise NotImplementedError("write your pallas kernel here")


# speedup vs baseline: 17.7154x; 17.7154x over previous
import functools

import jax
import jax.numpy as jnp
from jax import lax
from jax.experimental import pallas as pl
from jax.experimental.pallas import tpu as pltpu
from jax.experimental.pallas import tpu_sc as plsc

_LANES = 128
_SUB = 8
_CHUNK = _LANES * _SUB
_NW = 32


def _round_up(v, m):
    return (v + m - 1) // m * m


@functools.cache
def _sc_aggregate(n_pad, f, g_chunks):
    rows_per_sub = n_pad // 16
    mesh = plsc.VectorSubcoreMesh(core_axis_name="c", subcore_axis_name="s")

    @functools.partial(
        pl.kernel,
        mesh=mesh,
        out_type=jax.ShapeDtypeStruct((2, n_pad, f), jnp.float32),
        scratch_types=[
            pltpu.VMEM((_SUB, _LANES), jnp.int32),
            pltpu.VMEM((_SUB, _LANES), jnp.int32),
            pltpu.VMEM((_CHUNK, f), jnp.float32),
            pltpu.VMEM_SHARED((n_pad, f), jnp.float32),
            pltpu.SemaphoreType.DMA,
        ],
        compiler_params=pltpu.CompilerParams(use_tc_tiling_on_sc=False),
    )
    def agg(t_hbm, src_hbm, dst_hbm, zeros_hbm, out_hbm,
            src_v, dst_v, rows_v, accum, sem):
        c = lax.axis_index("c")
        s = lax.axis_index("s")
        w = s * 2 + c
        pltpu.sync_copy(zeros_hbm, accum.at[pl.ds(s * rows_per_sub, rows_per_sub)])
        plsc.subcore_barrier()

        def chunk(gi, carry):
            pltpu.sync_copy(src_hbm.at[w, gi], src_v)
            pltpu.sync_copy(dst_hbm.at[w, gi], dst_v)
            copies = [
                pltpu.async_copy(t_hbm.at[src_v.at[j]],
                                 rows_v.at[pl.ds(j * _LANES, _LANES)], sem)
                for j in range(_SUB)
            ]
            for cp in copies:
                cp.wait()
            for j in range(_SUB):
                pltpu.sync_copy(rows_v.at[pl.ds(j * _LANES, _LANES)],
                                accum.at[dst_v.at[j]], add=True)
            return carry

        lax.fori_loop(0, g_chunks, chunk, 0)
        plsc.subcore_barrier()
        pltpu.sync_copy(accum.at[pl.ds(s * rows_per_sub, rows_per_sub)],
                        out_hbm.at[c, pl.ds(s * rows_per_sub, rows_per_sub)])

    return agg


_DN = (((1,), (1,)), ((), ()))


def _tc_prep(x, wl, wr, b, f_out):
    n = x.shape[0]

    def body(x_ref, wl_ref, wr_ref, b_ref, t_ref, r_ref):
        xb = x_ref[...]
        t = lax.dot_general(xb, wl_ref[...], _DN, preferred_element_type=jnp.float32)
        col = lax.broadcasted_iota(jnp.int32, (n, f_out), 1)
        t_ref[...] = t + jnp.where(col == 24, 1.0, 0.0)
        r = lax.dot_general(xb, wr_ref[...], _DN, preferred_element_type=jnp.float32)
        r_ref[...] = r + b_ref[...]

    return pl.pallas_call(
        body,
        out_shape=[jax.ShapeDtypeStruct((n, f_out), jnp.float32)] * 2,
    )(x, wl, wr, b)


def _tc_mid(p, carry_in, wl, wr, b, f_in, f_out, invd_col, n):
    def body(p_ref, cin_ref, wl_ref, wr_ref, b_ref, t_ref, cout_ref):
        sums = p_ref[0, :n, :] + p_ref[1, :n, :]
        cin = cin_ref[...]
        if invd_col is None:
            invd = 1.0 / jnp.maximum(sums[:, 24:25], 1.0)
        else:
            invd = cin[:, invd_col:invd_col + 1]
        h = sums[:, :f_in] * invd + cin[:, :f_in]
        h = jnp.where(h > 0, h, 0.01 * h)
        t_ref[...] = lax.dot_general(h, wl_ref[...], _DN, preferred_element_type=jnp.float32)
        col = lax.broadcasted_iota(jnp.int32, (n, f_out), 1)
        r = lax.dot_general(h, wr_ref[...], _DN, preferred_element_type=jnp.float32) + b_ref[...]
        cout_ref[...] = jnp.where(col == f_out - 1, invd, r)

    return pl.pallas_call(
        body,
        out_shape=[jax.ShapeDtypeStruct((n, f_out), jnp.float32)] * 2,
    )(p, carry_in, wl, wr, b)


def _tc_final(p, carry_in, w_out, b_out, f_in, invd_col, n):
    def body(p_ref, cin_ref, w_ref, b_ref, o_ref):
        sums = p_ref[0, :n, :] + p_ref[1, :n, :]
        cin = cin_ref[...]
        invd = cin[:, invd_col:invd_col + 1]
        h = sums[:, :f_in] * invd + cin[:, :f_in]
        h = jnp.where(h > 0, h, 0.01 * h)
        logit = lax.dot_general(h, w_ref[...], _DN, preferred_element_type=jnp.float32) + b_ref[...]
        o_ref[...] = jax.nn.sigmoid(logit)

    return pl.pallas_call(
        body,
        out_shape=jax.ShapeDtypeStruct((n, 8), jnp.float32),
    )(p, carry_in, w_out, b_out)


def kernel(x, edge_index, W_l1, b_l1, W_r1, W_l2, b_l2, W_r2,
           W_l3, b_l3, W_r3, W_out, b_out):
    n, f_in = x.shape
    e = edge_index.shape[1]
    f1, f2, f3 = 32, 32, 16

    n_pad = _round_up(n, 256)
    if n_pad == n:
        n_pad += 256
    g_chunks = -(-e // (_NW * _CHUNK))
    e_pad = _NW * _CHUNK * g_chunks

    src = edge_index[0]
    dst = edge_index[1]
    pad = e_pad - e
    if pad:
        fill = jnp.arange(pad, dtype=jnp.int32)
        src = jnp.concatenate([src, fill % n])
        dst = jnp.concatenate([dst, n + fill % (n_pad - n)])
    src4 = src.reshape(_NW, g_chunks, _SUB, _LANES)
    dst4 = dst.reshape(_NW, g_chunks, _SUB, _LANES)
    zeros1 = jnp.zeros((n_pad // 16, f1), jnp.float32)
    zeros3 = jnp.zeros((n_pad // 16, f3), jnp.float32)

    def padw(mat, rows):
        return jnp.pad(mat, ((0, rows - mat.shape[0]), (0, 0)))

    def padb(vec, cols):
        return jnp.pad(vec, (0, cols - vec.shape[0])).reshape(1, cols)

    wl1, wr1, bl1 = padw(W_l1, f1), padw(W_r1, f1), padb(b_l1, f1)
    wl2, wr2, bl2 = padw(W_l2, f2), padw(W_r2, f2), padb(b_l2, f2)
    wl3, wr3, bl3 = padw(W_l3, f3), padw(W_r3, f3), padb(b_l3, f3)

    t1, r1 = _tc_prep(x, wl1, wr1, bl1, f1)
    agg_wide = _sc_aggregate(n_pad, f1, g_chunks)
    p1 = agg_wide(t1, src4, dst4, zeros1)
    t2, c2 = _tc_mid(p1, r1, wl2, wr2, bl2, 24, f2, None, n)
    p2 = agg_wide(t2, src4, dst4, zeros1)
    t3, c3 = _tc_mid(p2, c2, wl3, wr3, bl3, 18, f3, f2 - 1, n)
    agg_narrow = _sc_aggregate(n_pad, f3, g_chunks)
    p3 = agg_narrow(t3, src4, dst4, zeros3)
    w_out8 = jnp.pad(W_out, ((0, 8 - W_out.shape[0]), (0, 0)))
    b_out8 = jnp.pad(b_out, (0, 8 - b_out.shape[0])).reshape(1, 8)
    out = _tc_final(p3, c3, w_out8, b_out8, 8, f3 - 1, n)
    return out[:, 0]


# speedup vs baseline: 18.5791x; 1.0488x over previous
import functools

import jax
import jax.numpy as jnp
from jax import lax
from jax.experimental import pallas as pl
from jax.experimental.pallas import tpu as pltpu
from jax.experimental.pallas import tpu_sc as plsc

_LANES = 128
_SUB = 8
_CHUNK = _LANES * _SUB
_NW = 32


def _round_up(v, m):
    return (v + m - 1) // m * m


@functools.cache
def _sc_aggregate(n_pad, f, g_chunks):
    rows_per_sub = n_pad // 16
    mesh = plsc.VectorSubcoreMesh(core_axis_name="c", subcore_axis_name="s")

    @functools.partial(
        pl.kernel,
        mesh=mesh,
        out_type=jax.ShapeDtypeStruct((2, n_pad, f), jnp.float32),
        scratch_types=[
            pltpu.VMEM((_SUB, _LANES), jnp.int32),
            pltpu.VMEM((_SUB, _LANES), jnp.int32),
            pltpu.VMEM((_CHUNK, f), jnp.float32),
            pltpu.VMEM_SHARED((n_pad, f), jnp.float32),
            pltpu.SemaphoreType.DMA,
            pltpu.SemaphoreType.DMA,
        ],
        compiler_params=pltpu.CompilerParams(use_tc_tiling_on_sc=False),
    )
    def agg(t_hbm, src_hbm, dst_hbm, zeros_hbm, out_hbm,
            src_v, dst_v, rows_v, accum, sem, sem_s):
        c = lax.axis_index("c")
        s = lax.axis_index("s")
        w = s * 2 + c
        pltpu.sync_copy(zeros_hbm, accum.at[pl.ds(s * rows_per_sub, rows_per_sub)])
        plsc.subcore_barrier()

        def chunk(gi, carry):
            pltpu.sync_copy(src_hbm.at[w, gi], src_v)
            pltpu.sync_copy(dst_hbm.at[w, gi], dst_v)
            copies = [
                pltpu.async_copy(t_hbm.at[src_v.at[j]],
                                 rows_v.at[pl.ds(j * _LANES, _LANES)], sem)
                for j in range(_SUB)
            ]
            for cp in copies:
                cp.wait()
            scatters = [
                pltpu.async_copy(rows_v.at[pl.ds(j * _LANES, _LANES)],
                                 accum.at[dst_v.at[j]], sem_s, add=True)
                for j in range(_SUB)
            ]
            for cp in scatters:
                cp.wait()
            return carry

        lax.fori_loop(0, g_chunks, chunk, 0)
        plsc.subcore_barrier()
        pltpu.sync_copy(accum.at[pl.ds(s * rows_per_sub, rows_per_sub)],
                        out_hbm.at[c, pl.ds(s * rows_per_sub, rows_per_sub)])

    return agg


_DN = (((1,), (1,)), ((), ()))


def _tc_prep(x, wl, wr, b, f_out):
    n = x.shape[0]

    def body(x_ref, wl_ref, wr_ref, b_ref, t_ref, r_ref):
        xb = x_ref[...]
        t = lax.dot_general(xb, wl_ref[...], _DN, preferred_element_type=jnp.float32)
        col = lax.broadcasted_iota(jnp.int32, (n, f_out), 1)
        t_ref[...] = t + jnp.where(col == 24, 1.0, 0.0)
        r = lax.dot_general(xb, wr_ref[...], _DN, preferred_element_type=jnp.float32)
        r_ref[...] = r + b_ref[...]

    return pl.pallas_call(
        body,
        out_shape=[jax.ShapeDtypeStruct((n, f_out), jnp.float32)] * 2,
    )(x, wl, wr, b)


def _tc_mid(p, carry_in, wl, wr, b, f_in, f_out, invd_col, n):
    def body(p_ref, cin_ref, wl_ref, wr_ref, b_ref, t_ref, cout_ref):
        sums = p_ref[0, :n, :] + p_ref[1, :n, :]
        cin = cin_ref[...]
        if invd_col is None:
            invd = 1.0 / jnp.maximum(sums[:, 24:25], 1.0)
        else:
            invd = cin[:, invd_col:invd_col + 1]
        h = sums[:, :f_in] * invd + cin[:, :f_in]
        h = jnp.where(h > 0, h, 0.01 * h)
        t_ref[...] = lax.dot_general(h, wl_ref[...], _DN, preferred_element_type=jnp.float32)
        col = lax.broadcasted_iota(jnp.int32, (n, f_out), 1)
        r = lax.dot_general(h, wr_ref[...], _DN, preferred_element_type=jnp.float32) + b_ref[...]
        cout_ref[...] = jnp.where(col == f_out - 1, invd, r)

    return pl.pallas_call(
        body,
        out_shape=[jax.ShapeDtypeStruct((n, f_out), jnp.float32)] * 2,
    )(p, carry_in, wl, wr, b)


def _tc_final(p, carry_in, w_out, b_out, f_in, invd_col, n):
    def body(p_ref, cin_ref, w_ref, b_ref, o_ref):
        sums = p_ref[0, :n, :] + p_ref[1, :n, :]
        cin = cin_ref[...]
        invd = cin[:, invd_col:invd_col + 1]
        h = sums[:, :f_in] * invd + cin[:, :f_in]
        h = jnp.where(h > 0, h, 0.01 * h)
        logit = lax.dot_general(h, w_ref[...], _DN, preferred_element_type=jnp.float32) + b_ref[...]
        o_ref[...] = jax.nn.sigmoid(logit)

    return pl.pallas_call(
        body,
        out_shape=jax.ShapeDtypeStruct((n, 8), jnp.float32),
    )(p, carry_in, w_out, b_out)


def kernel(x, edge_index, W_l1, b_l1, W_r1, W_l2, b_l2, W_r2,
           W_l3, b_l3, W_r3, W_out, b_out):
    n, f_in = x.shape
    e = edge_index.shape[1]
    f1, f2, f3 = 32, 32, 16

    n_pad = _round_up(n, 256)
    if n_pad == n:
        n_pad += 256
    g_chunks = -(-e // (_NW * _CHUNK))
    e_pad = _NW * _CHUNK * g_chunks

    src = edge_index[0]
    dst = edge_index[1]
    pad = e_pad - e
    if pad:
        fill = jnp.arange(pad, dtype=jnp.int32)
        src = jnp.concatenate([src, fill % n])
        dst = jnp.concatenate([dst, n + fill % (n_pad - n)])
    src4 = src.reshape(_NW, g_chunks, _SUB, _LANES)
    dst4 = dst.reshape(_NW, g_chunks, _SUB, _LANES)
    zeros1 = jnp.zeros((n_pad // 16, f1), jnp.float32)
    zeros3 = jnp.zeros((n_pad // 16, f3), jnp.float32)

    def padw(mat, rows):
        return jnp.pad(mat, ((0, rows - mat.shape[0]), (0, 0)))

    def padb(vec, cols):
        return jnp.pad(vec, (0, cols - vec.shape[0])).reshape(1, cols)

    wl1, wr1, bl1 = padw(W_l1, f1), padw(W_r1, f1), padb(b_l1, f1)
    wl2, wr2, bl2 = padw(W_l2, f2), padw(W_r2, f2), padb(b_l2, f2)
    wl3, wr3, bl3 = padw(W_l3, f3), padw(W_r3, f3), padb(b_l3, f3)

    t1, r1 = _tc_prep(x, wl1, wr1, bl1, f1)
    agg_wide = _sc_aggregate(n_pad, f1, g_chunks)
    p1 = agg_wide(t1, src4, dst4, zeros1)
    t2, c2 = _tc_mid(p1, r1, wl2, wr2, bl2, 24, f2, None, n)
    p2 = agg_wide(t2, src4, dst4, zeros1)
    t3, c3 = _tc_mid(p2, c2, wl3, wr3, bl3, 18, f3, f2 - 1, n)
    agg_narrow = _sc_aggregate(n_pad, f3, g_chunks)
    p3 = agg_narrow(t3, src4, dst4, zeros3)
    w_out8 = jnp.pad(W_out, ((0, 8 - W_out.shape[0]), (0, 0)))
    b_out8 = jnp.pad(b_out, (0, 8 - b_out.shape[0])).reshape(1, 8)
    out = _tc_final(p3, c3, w_out8, b_out8, 8, f3 - 1, n)
    return out[:, 0]


# speedup vs baseline: 22.7284x; 1.2233x over previous
import functools

import jax
import jax.numpy as jnp
from jax import lax
from jax.experimental import pallas as pl
from jax.experimental.pallas import tpu as pltpu
from jax.experimental.pallas import tpu_sc as plsc

_LANES = 128
_SUB = 8
_CHUNK = _LANES * _SUB
_NW = 32


def _round_up(v, m):
    return (v + m - 1) // m * m


@functools.cache
def _sc_aggregate(n_pad, f, g_chunks):
    rows_per_sub = n_pad // 16
    mesh = plsc.VectorSubcoreMesh(core_axis_name="c", subcore_axis_name="s")

    @functools.partial(
        pl.kernel,
        mesh=mesh,
        out_type=jax.ShapeDtypeStruct((2, n_pad, f), jnp.float32),
        scratch_types=[
            pltpu.VMEM((2, _SUB, _LANES), jnp.int32),
            pltpu.VMEM((2, _SUB, _LANES), jnp.int32),
            pltpu.VMEM((2, _CHUNK, f), jnp.float32),
            pltpu.VMEM_SHARED((n_pad, f), jnp.float32),
            pltpu.SemaphoreType.DMA,
            pltpu.SemaphoreType.DMA,
        ],
        compiler_params=pltpu.CompilerParams(use_tc_tiling_on_sc=False),
    )
    def agg(t_hbm, src_hbm, dst_hbm, zeros_hbm, out_hbm,
            src_v, dst_v, rows_v, accum, sem_g, sem_s):
        c = lax.axis_index("c")
        s = lax.axis_index("s")
        w = s * 2 + c
        pltpu.sync_copy(zeros_hbm, accum.at[pl.ds(s * rows_per_sub, rows_per_sub)])
        plsc.subcore_barrier()

        def issue_gathers(buf, chunk_src):
            for j in range(_SUB):
                pltpu.async_copy(t_hbm.at[chunk_src.at[j]],
                                 buf.at[pl.ds(j * _LANES, _LANES)], sem_g)

        def drain(sem, ref):
            pltpu.make_async_copy(t_hbm.at[pl.ds(0, _CHUNK)], ref, sem).wait()

        pltpu.sync_copy(src_hbm.at[w, 0], src_v.at[0])
        pltpu.sync_copy(dst_hbm.at[w, 0], dst_v.at[0])
        issue_gathers(rows_v.at[0], src_v.at[0])

        def chunk(g, carry):
            b = jnp.bitwise_and(g, 1)
            nb = 1 - b

            @pl.when(g > 0)
            def _():
                drain(sem_s, rows_v.at[nb])

            @pl.when(g + 1 < g_chunks)
            def _():
                pltpu.sync_copy(src_hbm.at[w, g + 1], src_v.at[nb])
                pltpu.sync_copy(dst_hbm.at[w, g + 1], dst_v.at[nb])

            drain(sem_g, rows_v.at[b])

            @pl.when(g + 1 < g_chunks)
            def _():
                issue_gathers(rows_v.at[nb], src_v.at[nb])

            for j in range(_SUB):
                pltpu.async_copy(rows_v.at[b].at[pl.ds(j * _LANES, _LANES)],
                                 accum.at[dst_v.at[b].at[j]], sem_s, add=True)
            return carry

        lax.fori_loop(0, g_chunks, chunk, 0)
        drain(sem_s, rows_v.at[(g_chunks - 1) % 2])
        plsc.subcore_barrier()
        pltpu.sync_copy(accum.at[pl.ds(s * rows_per_sub, rows_per_sub)],
                        out_hbm.at[c, pl.ds(s * rows_per_sub, rows_per_sub)])

    return agg


_DN = (((1,), (1,)), ((), ()))


def _tc_prep(x, wl, wr, b, f_out):
    n = x.shape[0]

    def body(x_ref, wl_ref, wr_ref, b_ref, t_ref, r_ref):
        xb = x_ref[...]
        t = lax.dot_general(xb, wl_ref[...], _DN, preferred_element_type=jnp.float32)
        col = lax.broadcasted_iota(jnp.int32, (n, f_out), 1)
        t_ref[...] = t + jnp.where(col == 24, 1.0, 0.0)
        r = lax.dot_general(xb, wr_ref[...], _DN, preferred_element_type=jnp.float32)
        r_ref[...] = r + b_ref[...]

    return pl.pallas_call(
        body,
        out_shape=[jax.ShapeDtypeStruct((n, f_out), jnp.float32)] * 2,
    )(x, wl, wr, b)


def _tc_mid(p, carry_in, wl, wr, b, f_in, f_out, invd_col, n):
    def body(p_ref, cin_ref, wl_ref, wr_ref, b_ref, t_ref, cout_ref):
        sums = p_ref[0, :n, :] + p_ref[1, :n, :]
        cin = cin_ref[...]
        if invd_col is None:
            invd = 1.0 / jnp.maximum(sums[:, 24:25], 1.0)
        else:
            invd = cin[:, invd_col:invd_col + 1]
        h = sums[:, :f_in] * invd + cin[:, :f_in]
        h = jnp.where(h > 0, h, 0.01 * h)
        t_ref[...] = lax.dot_general(h, wl_ref[...], _DN, preferred_element_type=jnp.float32)
        col = lax.broadcasted_iota(jnp.int32, (n, f_out), 1)
        r = lax.dot_general(h, wr_ref[...], _DN, preferred_element_type=jnp.float32) + b_ref[...]
        cout_ref[...] = jnp.where(col == f_out - 1, invd, r)

    return pl.pallas_call(
        body,
        out_shape=[jax.ShapeDtypeStruct((n, f_out), jnp.float32)] * 2,
    )(p, carry_in, wl, wr, b)


def _tc_final(p, carry_in, w_out, b_out, f_in, invd_col, n):
    def body(p_ref, cin_ref, w_ref, b_ref, o_ref):
        sums = p_ref[0, :n, :] + p_ref[1, :n, :]
        cin = cin_ref[...]
        invd = cin[:, invd_col:invd_col + 1]
        h = sums[:, :f_in] * invd + cin[:, :f_in]
        h = jnp.where(h > 0, h, 0.01 * h)
        logit = lax.dot_general(h, w_ref[...], _DN, preferred_element_type=jnp.float32) + b_ref[...]
        o_ref[...] = jax.nn.sigmoid(logit)

    return pl.pallas_call(
        body,
        out_shape=jax.ShapeDtypeStruct((n, 8), jnp.float32),
    )(p, carry_in, w_out, b_out)


def kernel(x, edge_index, W_l1, b_l1, W_r1, W_l2, b_l2, W_r2,
           W_l3, b_l3, W_r3, W_out, b_out):
    n, f_in = x.shape
    e = edge_index.shape[1]
    f1, f2, f3 = 32, 32, 16

    n_pad = _round_up(n, 256)
    if n_pad == n:
        n_pad += 256
    g_chunks = -(-e // (_NW * _CHUNK))
    e_pad = _NW * _CHUNK * g_chunks

    src = edge_index[0]
    dst = edge_index[1]
    pad = e_pad - e
    if pad:
        fill = jnp.arange(pad, dtype=jnp.int32)
        src = jnp.concatenate([src, fill % n])
        dst = jnp.concatenate([dst, n + fill % (n_pad - n)])
    src4 = src.reshape(_NW, g_chunks, _SUB, _LANES)
    dst4 = dst.reshape(_NW, g_chunks, _SUB, _LANES)
    zeros1 = jnp.zeros((n_pad // 16, f1), jnp.float32)
    zeros3 = jnp.zeros((n_pad // 16, f3), jnp.float32)

    def padw(mat, rows):
        return jnp.pad(mat, ((0, rows - mat.shape[0]), (0, 0)))

    def padb(vec, cols):
        return jnp.pad(vec, (0, cols - vec.shape[0])).reshape(1, cols)

    wl1, wr1, bl1 = padw(W_l1, f1), padw(W_r1, f1), padb(b_l1, f1)
    wl2, wr2, bl2 = padw(W_l2, f2), padw(W_r2, f2), padb(b_l2, f2)
    wl3, wr3, bl3 = padw(W_l3, f3), padw(W_r3, f3), padb(b_l3, f3)

    t1, r1 = _tc_prep(x, wl1, wr1, bl1, f1)
    agg_wide = _sc_aggregate(n_pad, f1, g_chunks)
    p1 = agg_wide(t1, src4, dst4, zeros1)
    t2, c2 = _tc_mid(p1, r1, wl2, wr2, bl2, 24, f2, None, n)
    p2 = agg_wide(t2, src4, dst4, zeros1)
    t3, c3 = _tc_mid(p2, c2, wl3, wr3, bl3, 18, f3, f2 - 1, n)
    agg_narrow = _sc_aggregate(n_pad, f3, g_chunks)
    p3 = agg_narrow(t3, src4, dst4, zeros3)
    w_out8 = jnp.pad(W_out, ((0, 8 - W_out.shape[0]), (0, 0)))
    b_out8 = jnp.pad(b_out, (0, 8 - b_out.shape[0])).reshape(1, 8)
    out = _tc_final(p3, c3, w_out8, b_out8, 8, f3 - 1, n)
    return out[:, 0]


# speedup vs baseline: 23.2226x; 1.0217x over previous
import functools

import jax
import jax.numpy as jnp
from jax import lax
from jax.experimental import pallas as pl
from jax.experimental.pallas import tpu as pltpu
from jax.experimental.pallas import tpu_sc as plsc

_LANES = 128
_NW = 32


def _round_up(v, m):
    return (v + m - 1) // m * m


@functools.cache
def _sc_aggregate(n_pad, f, sub, g_chunks):
    chunk_e = sub * _LANES
    rows_per_sub = n_pad // 16
    mesh = plsc.VectorSubcoreMesh(core_axis_name="c", subcore_axis_name="s")

    @functools.partial(
        pl.kernel,
        mesh=mesh,
        out_type=jax.ShapeDtypeStruct((2, n_pad, f), jnp.float32),
        scratch_types=[
            pltpu.VMEM((2, sub, _LANES), jnp.int32),
            pltpu.VMEM((2, sub, _LANES), jnp.int32),
            pltpu.VMEM((2, chunk_e, f), jnp.float32),
            pltpu.VMEM_SHARED((n_pad, f), jnp.float32),
            pltpu.SemaphoreType.DMA,
            pltpu.SemaphoreType.DMA,
        ],
        compiler_params=pltpu.CompilerParams(use_tc_tiling_on_sc=False),
    )
    def agg(t_hbm, src_hbm, dst_hbm, zeros_hbm, out_hbm,
            src_v, dst_v, rows_v, accum, sem_g, sem_s):
        c = lax.axis_index("c")
        s = lax.axis_index("s")
        w = s * 2 + c
        pltpu.sync_copy(zeros_hbm, accum.at[pl.ds(s * rows_per_sub, rows_per_sub)])
        plsc.subcore_barrier()

        def issue_gathers(buf, chunk_src):
            for j in range(sub):
                pltpu.async_copy(t_hbm.at[chunk_src.at[j]],
                                 buf.at[pl.ds(j * _LANES, _LANES)], sem_g)

        def drain(sem, ref):
            pltpu.make_async_copy(t_hbm.at[pl.ds(0, chunk_e)], ref, sem).wait()

        pltpu.sync_copy(src_hbm.at[w, 0], src_v.at[0])
        pltpu.sync_copy(dst_hbm.at[w, 0], dst_v.at[0])
        issue_gathers(rows_v.at[0], src_v.at[0])

        def chunk(g, carry):
            b = jnp.bitwise_and(g, 1)
            nb = 1 - b

            @pl.when(g > 0)
            def _():
                drain(sem_s, rows_v.at[nb])

            @pl.when(g + 1 < g_chunks)
            def _():
                pltpu.sync_copy(src_hbm.at[w, g + 1], src_v.at[nb])
                pltpu.sync_copy(dst_hbm.at[w, g + 1], dst_v.at[nb])

            drain(sem_g, rows_v.at[b])

            @pl.when(g + 1 < g_chunks)
            def _():
                issue_gathers(rows_v.at[nb], src_v.at[nb])

            for j in range(sub):
                pltpu.async_copy(rows_v.at[b].at[pl.ds(j * _LANES, _LANES)],
                                 accum.at[dst_v.at[b].at[j]], sem_s, add=True)
            return carry

        lax.fori_loop(0, g_chunks, chunk, 0)
        drain(sem_s, rows_v.at[(g_chunks - 1) % 2])
        plsc.subcore_barrier()
        pltpu.sync_copy(accum.at[pl.ds(s * rows_per_sub, rows_per_sub)],
                        out_hbm.at[c, pl.ds(s * rows_per_sub, rows_per_sub)])

    return agg


_DN = (((1,), (1,)), ((), ()))


def _tc_prep(x, wl, wr, b, f_out):
    n = x.shape[0]

    def body(x_ref, wl_ref, wr_ref, b_ref, t_ref, r_ref):
        xb = x_ref[...]
        t = lax.dot_general(xb, wl_ref[...], _DN, preferred_element_type=jnp.float32)
        col = lax.broadcasted_iota(jnp.int32, (n, f_out), 1)
        t_ref[...] = t + jnp.where(col == 24, 1.0, 0.0)
        r = lax.dot_general(xb, wr_ref[...], _DN, preferred_element_type=jnp.float32)
        r_ref[...] = r + b_ref[...]

    return pl.pallas_call(
        body,
        out_shape=[jax.ShapeDtypeStruct((n, f_out), jnp.float32)] * 2,
    )(x, wl, wr, b)


def _tc_mid(p, carry_in, wl, wr, b, f_in, f_out, invd_col, n):
    def body(p_ref, cin_ref, wl_ref, wr_ref, b_ref, t_ref, cout_ref):
        sums = p_ref[0, :n, :] + p_ref[1, :n, :]
        cin = cin_ref[...]
        if invd_col is None:
            invd = 1.0 / jnp.maximum(sums[:, 24:25], 1.0)
        else:
            invd = cin[:, invd_col:invd_col + 1]
        h = sums[:, :f_in] * invd + cin[:, :f_in]
        h = jnp.where(h > 0, h, 0.01 * h)
        t_ref[...] = lax.dot_general(h, wl_ref[...], _DN, preferred_element_type=jnp.float32)
        col = lax.broadcasted_iota(jnp.int32, (n, f_out), 1)
        r = lax.dot_general(h, wr_ref[...], _DN, preferred_element_type=jnp.float32) + b_ref[...]
        cout_ref[...] = jnp.where(col == f_out - 1, invd, r)

    return pl.pallas_call(
        body,
        out_shape=[jax.ShapeDtypeStruct((n, f_out), jnp.float32)] * 2,
    )(p, carry_in, wl, wr, b)


def _tc_final(p, carry_in, w_out, b_out, f_in, invd_col, n):
    def body(p_ref, cin_ref, w_ref, b_ref, o_ref):
        sums = p_ref[0, :n, :] + p_ref[1, :n, :]
        cin = cin_ref[...]
        invd = cin[:, invd_col:invd_col + 1]
        h = sums[:, :f_in] * invd + cin[:, :f_in]
        h = jnp.where(h > 0, h, 0.01 * h)
        logit = lax.dot_general(h, w_ref[...], _DN, preferred_element_type=jnp.float32) + b_ref[...]
        o_ref[...] = jax.nn.sigmoid(logit)

    return pl.pallas_call(
        body,
        out_shape=jax.ShapeDtypeStruct((n, 8), jnp.float32),
    )(p, carry_in, w_out, b_out)


def kernel(x, edge_index, W_l1, b_l1, W_r1, W_l2, b_l2, W_r2,
           W_l3, b_l3, W_r3, W_out, b_out):
    n, f_in = x.shape
    e = edge_index.shape[1]
    f1, f2, f3 = 32, 32, 16

    n_pad = _round_up(n, 256)
    if n_pad == n:
        n_pad += 256
    sub_a, sub_b = 10, 16
    stride = _NW * _LANES * sub_a * sub_b // 2
    e_pad = _round_up(e, stride)
    g_a = e_pad // (_NW * _LANES * sub_a)
    g_b = e_pad // (_NW * _LANES * sub_b)

    src = edge_index[0]
    dst = edge_index[1]
    pad = e_pad - e
    if pad:
        fill = jnp.arange(pad, dtype=jnp.int32)
        src = jnp.concatenate([src, fill % n])
        dst = jnp.concatenate([dst, n + fill % (n_pad - n)])
    src_a = src.reshape(_NW, g_a, sub_a, _LANES)
    dst_a = dst.reshape(_NW, g_a, sub_a, _LANES)
    src_b = src.reshape(_NW, g_b, sub_b, _LANES)
    dst_b = dst.reshape(_NW, g_b, sub_b, _LANES)
    zeros1 = jnp.zeros((n_pad // 16, f1), jnp.float32)
    zeros3 = jnp.zeros((n_pad // 16, f3), jnp.float32)

    def padw(mat, rows):
        return jnp.pad(mat, ((0, rows - mat.shape[0]), (0, 0)))

    def padb(vec, cols):
        return jnp.pad(vec, (0, cols - vec.shape[0])).reshape(1, cols)

    wl1, wr1, bl1 = padw(W_l1, f1), padw(W_r1, f1), padb(b_l1, f1)
    wl2, wr2, bl2 = padw(W_l2, f2), padw(W_r2, f2), padb(b_l2, f2)
    wl3, wr3, bl3 = padw(W_l3, f3), padw(W_r3, f3), padb(b_l3, f3)

    t1, r1 = _tc_prep(x, wl1, wr1, bl1, f1)
    agg_wide = _sc_aggregate(n_pad, f1, sub_a, g_a)
    p1 = agg_wide(t1, src_a, dst_a, zeros1)
    t2, c2 = _tc_mid(p1, r1, wl2, wr2, bl2, 24, f2, None, n)
    p2 = agg_wide(t2, src_a, dst_a, zeros1)
    t3, c3 = _tc_mid(p2, c2, wl3, wr3, bl3, 18, f3, f2 - 1, n)
    agg_narrow = _sc_aggregate(n_pad, f3, sub_b, g_b)
    p3 = agg_narrow(t3, src_b, dst_b, zeros3)
    w_out8 = jnp.pad(W_out, ((0, 8 - W_out.shape[0]), (0, 0)))
    b_out8 = jnp.pad(b_out, (0, 8 - b_out.shape[0])).reshape(1, 8)
    out = _tc_final(p3, c3, w_out8, b_out8, 8, f3 - 1, n)
    return out[:, 0]


# speedup vs baseline: 27.8163x; 1.1978x over previous
import functools

import jax
import jax.numpy as jnp
from jax import lax
from jax.experimental import pallas as pl
from jax.experimental.pallas import tpu as pltpu
from jax.experimental.pallas import tpu_sc as plsc

_LANES = 128
_SUB = 10
_NW = 32
_F = 32


def _round_up(v, m):
    return (v + m - 1) // m * m


@functools.cache
def _sc_aggregate(n_pad, g_chunks):
    chunk_e = _SUB * _LANES
    rows_per_sub = n_pad // 16
    mesh = plsc.VectorSubcoreMesh(core_axis_name="c", subcore_axis_name="s")

    @functools.partial(
        pl.kernel,
        mesh=mesh,
        out_type=jax.ShapeDtypeStruct((2, n_pad, _F), jnp.float32),
        scratch_types=[
            pltpu.VMEM((2, _SUB, _LANES), jnp.int32),
            pltpu.VMEM((2, _SUB, _LANES), jnp.int32),
            pltpu.VMEM((2, chunk_e, _F), jnp.float32),
            pltpu.VMEM_SHARED((n_pad, _F), jnp.float32),
            pltpu.SemaphoreType.DMA,
            pltpu.SemaphoreType.DMA,
        ],
        compiler_params=pltpu.CompilerParams(use_tc_tiling_on_sc=False),
    )
    def agg(t_hbm, src_hbm, dst_hbm, zeros_hbm, out_hbm,
            src_v, dst_v, rows_v, accum, sem_g, sem_s):
        c = lax.axis_index("c")
        s = lax.axis_index("s")
        w = s * 2 + c
        pltpu.sync_copy(zeros_hbm, accum.at[pl.ds(s * rows_per_sub, rows_per_sub)])
        plsc.subcore_barrier()

        def issue_gathers(buf, chunk_src):
            for j in range(_SUB):
                pltpu.async_copy(t_hbm.at[chunk_src.at[j]],
                                 buf.at[pl.ds(j * _LANES, _LANES)], sem_g)

        def drain(sem, ref):
            pltpu.make_async_copy(t_hbm.at[pl.ds(0, chunk_e)], ref, sem).wait()

        pltpu.sync_copy(src_hbm.at[w, 0], src_v.at[0])
        pltpu.sync_copy(dst_hbm.at[w, 0], dst_v.at[0])
        issue_gathers(rows_v.at[0], src_v.at[0])

        def chunk(g, carry):
            b = jnp.bitwise_and(g, 1)
            nb = 1 - b

            @pl.when(g > 0)
            def _():
                drain(sem_s, rows_v.at[nb])

            @pl.when(g + 1 < g_chunks)
            def _():
                pltpu.sync_copy(src_hbm.at[w, g + 1], src_v.at[nb])
                pltpu.sync_copy(dst_hbm.at[w, g + 1], dst_v.at[nb])

            drain(sem_g, rows_v.at[b])

            @pl.when(g + 1 < g_chunks)
            def _():
                issue_gathers(rows_v.at[nb], src_v.at[nb])

            for j in range(_SUB):
                pltpu.async_copy(rows_v.at[b].at[pl.ds(j * _LANES, _LANES)],
                                 accum.at[dst_v.at[b].at[j]], sem_s, add=True)
            return carry

        lax.fori_loop(0, g_chunks, chunk, 0)
        drain(sem_s, rows_v.at[(g_chunks - 1) % 2])
        plsc.subcore_barrier()
        pltpu.sync_copy(accum.at[pl.ds(s * rows_per_sub, rows_per_sub)],
                        out_hbm.at[c, pl.ds(s * rows_per_sub, rows_per_sub)])

    return agg


_DN = (((1,), (1,)), ((), ()))
_DOT = (((1,), (0,)), ((), ()))


def _tc_prep(x, wl, wr, b):
    n = x.shape[0]

    def body(x_ref, wl_ref, wr_ref, b_ref, t_ref, r_ref):
        xb = x_ref[...]
        t = lax.dot_general(xb, wl_ref[...], _DN, preferred_element_type=jnp.float32)
        col = lax.broadcasted_iota(jnp.int32, (n, _F), 1)
        t_ref[...] = t + jnp.where(col == 24, 1.0, 0.0)
        r = lax.dot_general(xb, wr_ref[...], _DN, preferred_element_type=jnp.float32)
        r_ref[...] = r + b_ref[...]

    return pl.pallas_call(
        body,
        out_shape=[jax.ShapeDtypeStruct((n, _F), jnp.float32)] * 2,
    )(x, wl, wr, b)


def _tc_mid(p, cin, bdl, bdr, sel, b_tiled, m31, first):
    r_rows = p.shape[1]

    def body(p_ref, cin_ref, bdl_ref, bdr_ref, sel_ref, bt_ref, m31_ref,
             t_ref, cout_ref):
        sums = p_ref[0] + p_ref[1]
        cinb = cin_ref[...]
        if first:
            deg = lax.dot_general(sums, sel_ref[...], _DOT,
                                  preferred_element_type=jnp.float32)
            invd = 1.0 / jnp.maximum(deg, 1.0)
        else:
            invd = lax.dot_general(cinb, sel_ref[...], _DOT,
                                   preferred_element_type=jnp.float32)
        h = sums * invd + cinb
        h = jnp.where(h > 0, h, 0.01 * h)
        t_ref[...] = lax.dot_general(h, bdl_ref[...], _DOT,
                                     preferred_element_type=jnp.float32)
        cout = lax.dot_general(h, bdr_ref[...], _DOT,
                               preferred_element_type=jnp.float32) + bt_ref[...]
        m31 = m31_ref[...]
        cout_ref[...] = cout * (1.0 - m31) + invd * m31

    return pl.pallas_call(
        body,
        out_shape=[jax.ShapeDtypeStruct((r_rows, 128), jnp.float32)] * 2,
    )(p, cin, bdl, bdr, sel, b_tiled, m31)


def _tc_final(p, cin, bd_out, sel, b_tiled):
    r_rows = p.shape[1]

    def body(p_ref, cin_ref, bd_ref, sel_ref, bt_ref, o_ref):
        sums = p_ref[0] + p_ref[1]
        cinb = cin_ref[...]
        invd = lax.dot_general(cinb, sel_ref[...], _DOT,
                               preferred_element_type=jnp.float32)
        h = sums * invd + cinb
        h = jnp.where(h > 0, h, 0.01 * h)
        logit = lax.dot_general(h, bd_ref[...], _DOT,
                                preferred_element_type=jnp.float32) + bt_ref[...]
        o_ref[...] = jax.nn.sigmoid(logit)

    return pl.pallas_call(
        body,
        out_shape=jax.ShapeDtypeStruct((r_rows, 128), jnp.float32),
    )(p, cin, bd_out, sel, b_tiled)


def kernel(x, edge_index, W_l1, b_l1, W_r1, W_l2, b_l2, W_r2,
           W_l3, b_l3, W_r3, W_out, b_out):
    n, f_in = x.shape
    e = edge_index.shape[1]
    f32 = jnp.float32

    n_pad = _round_up(n, 256)
    if n_pad == n:
        n_pad += 256
    stride = _NW * _LANES * _SUB
    e_pad = _round_up(e, stride)
    g_chunks = e_pad // stride

    src = edge_index[0]
    dst = edge_index[1]
    pad = e_pad - e
    if pad:
        fill = jnp.arange(pad, dtype=jnp.int32)
        src = jnp.concatenate([src, fill % n])
        dst = jnp.concatenate([dst, n + fill % (n_pad - n)])
    src4 = src.reshape(_NW, g_chunks, _SUB, _LANES)
    dst4 = dst.reshape(_NW, g_chunks, _SUB, _LANES)
    zeros = jnp.zeros((n_pad // 16, _F), f32)

    def padw(mat):
        return jnp.pad(mat, ((0, _F - mat.shape[0]), (0, 0)))

    def padb(vec):
        return jnp.pad(vec, (0, _F - vec.shape[0])).reshape(1, _F)

    def bd(mat):
        blk = jnp.pad(mat, ((0, _F - mat.shape[0]), (0, _F - mat.shape[1]))).T
        return jnp.kron(jnp.eye(4, dtype=f32), blk)

    def tile4(row):
        return jnp.tile(row, (1, 4))

    lane = jnp.arange(128)
    m31 = ((lane % _F) == _F - 1).astype(f32).reshape(1, 128)
    s24 = (lane[:, None] == (lane[None, :] // _F) * _F + 24).astype(f32)
    s31 = (lane[:, None] == (lane[None, :] // _F) * _F + (_F - 1)).astype(f32)

    t1, r1 = _tc_prep(x, padw(W_l1), padw(W_r1), padb(b_l1))
    rpad = ((0, n_pad - n), (0, 0))
    t1 = jnp.pad(t1, rpad)
    r1p = jnp.pad(r1, rpad).reshape(n_pad // 4, 128)

    def unpack(a):
        return a.reshape(n_pad, _F)

    def repack(a):
        return a.reshape(2, n_pad // 4, 128)

    agg = _sc_aggregate(n_pad, g_chunks)

    p1 = agg(t1, src4, dst4, zeros)
    t2p, c2p = _tc_mid(repack(p1), r1p, bd(W_l2), bd(W_r2), s24,
                       tile4(padb(b_l2)), m31, first=True)
    p2 = agg(unpack(t2p), src4, dst4, zeros)
    t3p, c3p = _tc_mid(repack(p2), c2p, bd(W_l3), bd(W_r3), s31,
                       tile4(padb(b_l3)), m31, first=False)
    p3 = agg(unpack(t3p), src4, dst4, zeros)
    outp = _tc_final(repack(p3), c3p, bd(W_out), s31,
                     tile4(padb(b_out)))
    return outp.reshape(n_pad, _F)[:n, 0]
